# baseline (speedup 1.0000x reference)
import functools

import jax
import jax.numpy as jnp
from jax import lax
from jax.experimental import pallas as pl
from jax.experimental.pallas import tpu as pltpu
from jax.experimental.pallas import tpu_sc as plsc

NUM_TOKENS = 4096 * 200
DIM = 128

_CHUNK = 64
_NBUF = 5
_NSLOT = 4
_UNROLL = 20


def _build():
    info = plsc.get_sparse_core_info()
    nw = info.num_cores * info.num_subcores
    rows_per_w = NUM_TOKENS // nw
    n_chunks = rows_per_w // _CHUNK
    n_groups = n_chunks // _UNROLL
    idx_rows_per_w = n_chunks

    mesh = plsc.VectorSubcoreMesh(core_axis_name="c", subcore_axis_name="s")

    @functools.partial(
        pl.kernel,
        mesh=mesh,
        out_type=jax.ShapeDtypeStruct((NUM_TOKENS, DIM), jnp.float32),
        scratch_types=[
            pltpu.VMEM((idx_rows_per_w, _CHUNK), jnp.int32),
            pltpu.VMEM((_NBUF, _CHUNK, DIM), jnp.float32),
            pltpu.VMEM_SHARED((info.num_subcores, _NSLOT, _CHUNK, DIM),
                              jnp.float32),
        ] + [pltpu.SemaphoreType.DMA] * (_NBUF + 2 * _NSLOT),
    )
    def emb(idx_hbm, table_hbm, out_hbm, idx_v, rows_v, sp, *sems):
        gsems = sems[:_NBUF]
        csems = sems[_NBUF:_NBUF + _NSLOT]
        hsems = sems[_NBUF + _NSLOT:]

        wid = lax.axis_index("s") * info.num_cores + lax.axis_index("c")
        sid = lax.axis_index("s")
        base = wid * rows_per_w

        pltpu.sync_copy(idx_hbm.at[pl.ds(wid * idx_rows_per_w, idx_rows_per_w)],
                        idx_v)

        def out_at(j):
            return out_hbm.at[pl.ds(base + j * _CHUNK, _CHUNK)]

        def gather(j, b):
            return pltpu.make_async_copy(
                table_hbm.at[idx_v.at[j]], rows_v.at[b], gsems[b])

        def xbar(b, s):
            return pltpu.make_async_copy(rows_v.at[b], sp.at[sid, s], csems[s])

        def drain_sp(j, s):
            return pltpu.make_async_copy(sp.at[sid, s], out_at(j), hsems[s])

        for b in range(_NBUF):
            gather(b, b).start()

        def do_group(j0, first, last):
            for c in range(_UNROLL):
                j = j0 + c
                b = c % _NBUF
                s = c % _NSLOT
                gather(j, b).wait()
                if not first or c >= _NSLOT:
                    drain_sp(j - _NSLOT, s).wait()
                xbar(b, s).start()
                xbar(b, s).wait()
                drain_sp(j, s).start()
                if not last or c < _UNROLL - _NBUF:
                    gather(j + _NBUF, b).start()

        do_group(0, first=True, last=False)

        def group(g, _):
            do_group(g * _UNROLL, first=False, last=False)
            return _

        lax.fori_loop(1, n_groups - 1, group, None)

        do_group((n_groups - 1) * _UNROLL, first=False, last=True)

        for k in range(_NSLOT):
            j = n_chunks - _NSLOT + k
            drain_sp(j, j % _NSLOT).wait()

    return emb


_EMB = _build()


@jax.jit
def kernel(token_ids, weight):
    idx2d = token_ids.reshape(NUM_TOKENS // _CHUNK, _CHUNK).astype(jnp.int32)
    out = _EMB(idx2d, weight)
    return out.reshape(*token_ids.shape, DIM)


# speedup vs baseline: 1.0018x; 1.0018x over previous
import functools

import jax
import jax.numpy as jnp
from jax import lax
from jax.experimental import pallas as pl
from jax.experimental.pallas import tpu as pltpu
from jax.experimental.pallas import tpu_sc as plsc

NUM_TOKENS = 4096 * 200
DIM = 128

_CHUNK = 80
_NBUF = 4
_NSLOT = 4
_UNROLL = 20


def _build():
    info = plsc.get_sparse_core_info()
    nw = info.num_cores * info.num_subcores
    rows_per_w = NUM_TOKENS // nw
    n_chunks = rows_per_w // _CHUNK
    n_groups = n_chunks // _UNROLL
    idx_rows_per_w = n_chunks

    mesh = plsc.VectorSubcoreMesh(core_axis_name="c", subcore_axis_name="s")

    @functools.partial(
        pl.kernel,
        mesh=mesh,
        out_type=jax.ShapeDtypeStruct((NUM_TOKENS, DIM), jnp.float32),
        scratch_types=[
            pltpu.VMEM((idx_rows_per_w, _CHUNK), jnp.int32),
            pltpu.VMEM((_NBUF, _CHUNK, DIM), jnp.float32),
            pltpu.VMEM_SHARED((info.num_subcores, _NSLOT, _CHUNK, DIM),
                              jnp.float32),
        ] + [pltpu.SemaphoreType.DMA] * (_NBUF + 2 * _NSLOT),
    )
    def emb(idx_hbm, table_hbm, out_hbm, idx_v, rows_v, sp, *sems):
        gsems = sems[:_NBUF]
        csems = sems[_NBUF:_NBUF + _NSLOT]
        hsems = sems[_NBUF + _NSLOT:]

        wid = lax.axis_index("s") * info.num_cores + lax.axis_index("c")
        sid = lax.axis_index("s")
        base = wid * rows_per_w

        pltpu.sync_copy(idx_hbm.at[pl.ds(wid * idx_rows_per_w, idx_rows_per_w)],
                        idx_v)

        def out_at(j):
            return out_hbm.at[pl.ds(base + j * _CHUNK, _CHUNK)]

        def gather(j, b):
            return pltpu.make_async_copy(
                table_hbm.at[idx_v.at[j]], rows_v.at[b], gsems[b])

        def xbar(b, s):
            return pltpu.make_async_copy(rows_v.at[b], sp.at[sid, s], csems[s])

        def drain_sp(j, s):
            return pltpu.make_async_copy(sp.at[sid, s], out_at(j), hsems[s])

        for b in range(_NBUF):
            gather(b, b).start()

        def finish(jp, bp, sp_, last):
            xbar(bp, sp_).wait()
            drain_sp(jp, sp_).start()
            if not last or (jp % _UNROLL) < _UNROLL - _NBUF:
                gather(jp + _NBUF, bp).start()

        def do_group(j0, first, last):
            pend = None
            for c in range(_UNROLL):
                j = j0 + c
                b = c % _NBUF
                s = c % _NSLOT
                gather(j, b).wait()
                if not first or c >= _NSLOT:
                    drain_sp(j - _NSLOT, s).wait()
                xbar(b, s).start()
                if pend is not None:
                    finish(*pend, last)
                pend = (j, b, s)
            finish(*pend, last)

        do_group(0, first=True, last=False)

        def group(g, _):
            do_group(g * _UNROLL, first=False, last=False)
            return _

        lax.fori_loop(1, n_groups - 1, group, None)

        do_group((n_groups - 1) * _UNROLL, first=False, last=True)

        for k in range(_NSLOT):
            j = n_chunks - _NSLOT + k
            drain_sp(j, j % _NSLOT).wait()

    return emb


_EMB = _build()


@jax.jit
def kernel(token_ids, weight):
    idx2d = token_ids.reshape(NUM_TOKENS // _CHUNK, _CHUNK).astype(jnp.int32)
    out = _EMB(idx2d, weight)
    return out.reshape(*token_ids.shape, DIM)


# speedup vs baseline: 1.0125x; 1.0107x over previous
import functools

import jax
import jax.numpy as jnp
from jax import lax
from jax.experimental import pallas as pl
from jax.experimental.pallas import tpu as pltpu
from jax.experimental.pallas import tpu_sc as plsc

NUM_TOKENS = 4096 * 200
DIM = 128

_CHUNK = 80
_CYCLE = 4


def _build():
    info = plsc.get_sparse_core_info()
    nw = info.num_cores * info.num_subcores
    rows_per_w = NUM_TOKENS // nw
    n_chunks = rows_per_w // _CHUNK
    n_groups = n_chunks // _CYCLE
    idx_rows_per_w = n_chunks

    mesh = plsc.VectorSubcoreMesh(core_axis_name="c", subcore_axis_name="s")

    @functools.partial(
        pl.kernel,
        mesh=mesh,
        out_type=jax.ShapeDtypeStruct((NUM_TOKENS, DIM), jnp.float32),
        scratch_types=[
            pltpu.VMEM((idx_rows_per_w, _CHUNK), jnp.int32),
            pltpu.VMEM((_CYCLE, _CHUNK, DIM), jnp.float32),
            pltpu.VMEM_SHARED((info.num_subcores, _CYCLE, _CHUNK, DIM),
                              jnp.float32),
        ] + [pltpu.SemaphoreType.DMA] * (3 * _CYCLE),
    )
    def emb(idx_hbm, table_hbm, out_hbm, idx_v, rows_v, sp, *sems):
        gsems = sems[:_CYCLE]
        csems = sems[_CYCLE:2 * _CYCLE]
        hsems = sems[2 * _CYCLE:]

        wid = lax.axis_index("s") * info.num_cores + lax.axis_index("c")
        sid = lax.axis_index("s")
        base = wid * rows_per_w

        pltpu.sync_copy(idx_hbm.at[pl.ds(wid * idx_rows_per_w, idx_rows_per_w)],
                        idx_v)

        def out_at(j):
            return out_hbm.at[pl.ds(base + j * _CHUNK, _CHUNK)]

        def gather(j, b):
            return pltpu.make_async_copy(
                table_hbm.at[idx_v.at[j]], rows_v.at[b], gsems[b])

        def xbar(k):
            return pltpu.make_async_copy(rows_v.at[k], sp.at[sid, k], csems[k])

        def drain_sp(j, k):
            return pltpu.make_async_copy(sp.at[sid, k], out_at(j), hsems[k])

        for k in range(_CYCLE):
            gather(k, k).start()

        def do_group(j0, first, last):
            for k in range(_CYCLE):
                j = j0 + k
                gather(j, k).wait()
                if not first:
                    drain_sp(j - _CYCLE, k).wait()
                xbar(k).start()
                xbar(k).wait()
                drain_sp(j, k).start()
                if not last:
                    gather(j + _CYCLE, k).start()

        do_group(0, first=True, last=False)

        def group(g, _):
            do_group(g * _CYCLE, first=False, last=False)
            return _

        lax.fori_loop(1, n_groups - 1, group, None)

        do_group((n_groups - 1) * _CYCLE, first=False, last=True)

        j0 = (n_groups - 1) * _CYCLE
        for k in range(_CYCLE):
            drain_sp(j0 + k, k).wait()

    return emb


_EMB = _build()


@jax.jit
def kernel(token_ids, weight):
    idx2d = token_ids.reshape(NUM_TOKENS // _CHUNK, _CHUNK).astype(jnp.int32)
    out = _EMB(idx2d, weight)
    return out.reshape(*token_ids.shape, DIM)
